# Initial kernel scaffold; baseline (speedup 1.0000x reference)
import jax
import jax.numpy as jnp
from jax.experimental import pallas as pl


def kernel(features, roi):
    raise NotImplementedError("write your pallas kernel here")


# speedup vs baseline: 2.2250x; 2.2250x over previous
import functools

import jax
import jax.numpy as jnp
from jax import lax
from jax.experimental import pallas as pl
from jax.experimental.pallas import tpu as pltpu
from jax.experimental.pallas import tpu_sc as plsc

_P = 7
_RPB = 4
_NW = 32


def _index_kernel(roi_ref, idx_ref, w_ref, *, H, W):
    roi = roi_ref[...]
    y1 = roi[:, 0][None, :]
    x1 = roi[:, 1][None, :]
    y2 = roi[:, 2][None, :]
    x2 = roi[:, 3][None, :]

    h = jnp.maximum(x2 - x1, 1)
    w = jnp.maximum(y2 - y1, 1)
    iv = lax.broadcasted_iota(jnp.int32, (_P, 1), 0).astype(jnp.float32)

    rpos = iv * (h.astype(jnp.float32) / _P)
    r0 = jnp.floor(rpos).astype(jnp.int32)
    r1 = jnp.minimum(r0 + 1, h - 1)
    rf = rpos - r0.astype(jnp.float32)
    R0 = jnp.clip(x1 + r0, 0, H - 1)
    R1 = jnp.clip(x1 + r1, 0, H - 1)

    cpos = iv * (w.astype(jnp.float32) / _P)
    c0 = jnp.floor(cpos).astype(jnp.int32)
    c1 = jnp.minimum(c0 + 1, w - 1)
    cf = cpos - c0.astype(jnp.float32)
    C0 = jnp.clip(y1 + c0, 0, W - 1)
    C1 = jnp.clip(y1 + c1, 0, W - 1)

    R0b = R0[:, None, :] * W
    R1b = R1[:, None, :] * W
    C0b = C0[None, :, :]
    C1b = C1[None, :, :]
    idx_ref[0] = R0b + C0b
    idx_ref[1] = R0b + C1b
    idx_ref[2] = R1b + C0b
    idx_ref[3] = R1b + C1b

    rfb = rf[:, None, :]
    cfb = cf[None, :, :]
    one = jnp.float32(1.0)
    w_ref[0] = (one - rfb) * (one - cfb)
    w_ref[1] = (one - rfb) * cfb
    w_ref[2] = rfb * (one - cfb)
    w_ref[3] = rfb * cfb


def _make_sc_kernel(C, NB):
    npos = _P * _P
    rows_per_block = _RPB * npos
    mesh = plsc.VectorSubcoreMesh(core_axis_name="c", subcore_axis_name="s")
    info = plsc.get_sparse_core_info()
    nc = info.num_cores

    @functools.partial(
        pl.kernel,
        mesh=mesh,
        out_type=jax.ShapeDtypeStruct((NB * rows_per_block, C), jnp.float32),
        scratch_types=[
            pltpu.VMEM((npos, 16), jnp.int32),
            pltpu.VMEM((npos, 16), jnp.float32),
            pltpu.VMEM((4 * _RPB, C), jnp.float32),
            pltpu.VMEM((rows_per_block, C), jnp.float32),
            pltpu.SemaphoreType.DMA,
        ],
        compiler_params=pltpu.CompilerParams(use_tc_tiling_on_sc=False),
    )
    def sc_kernel(feat_hbm, idx_hbm, w_hbm, out_hbm, idx_v, w_v, gbuf, obuf, sem):
        wid = lax.axis_index("s") * nc + lax.axis_index("c")
        nblk = (NB - wid + _NW - 1) // _NW

        def outer(i, carry):
            b = wid + i * _NW
            pltpu.sync_copy(idx_hbm.at[b], idx_v)
            pltpu.sync_copy(w_hbm.at[b], w_v)

            def inner(p, carry2):
                iv = idx_v[p]
                pltpu.async_copy(feat_hbm.at[iv], gbuf, sem).wait()
                wrow = w_v[p]
                for r in range(_RPB):
                    w00 = wrow[r]
                    w01 = wrow[4 + r]
                    w10 = wrow[8 + r]
                    w11 = wrow[12 + r]
                    orow = r * npos + p

                    def cbody(c, carry3):
                        cs = pl.ds(c * 16, 16)
                        obuf[orow, cs] = (w00 * gbuf[r, cs]
                                          + w01 * gbuf[4 + r, cs]
                                          + w10 * gbuf[8 + r, cs]
                                          + w11 * gbuf[12 + r, cs])
                        return carry3

                    lax.fori_loop(0, C // 16, cbody, 0)
                return carry2

            lax.fori_loop(0, npos, inner, 0)
            pltpu.sync_copy(obuf, out_hbm.at[pl.ds(b * rows_per_block, rows_per_block)])
            return carry

        lax.fori_loop(0, nblk, outer, 0)

    return sc_kernel


def kernel(features, roi):
    B, H, W, C = features.shape
    N = roi.shape[0] * roi.shape[1]
    roi2 = roi.reshape(N, 4).astype(jnp.int32)
    feat2 = features.reshape(B * H * W, C)

    idx4, w4 = pl.pallas_call(
        functools.partial(_index_kernel, H=H, W=W),
        out_shape=[
            jax.ShapeDtypeStruct((4, _P, _P, N), jnp.int32),
            jax.ShapeDtypeStruct((4, _P, _P, N), jnp.float32),
        ],
    )(roi2)

    NB = N // _RPB
    npos = _P * _P
    idx_b = (idx4.reshape(4, npos, NB, _RPB)
             .transpose(2, 1, 0, 3)
             .reshape(NB, npos, 16))
    w_b = (w4.reshape(4, npos, NB, _RPB)
           .transpose(2, 1, 0, 3)
           .reshape(NB, npos, 16))

    out = _make_sc_kernel(C, NB)(feat2, idx_b, w_b)
    return out.reshape(N, _P, _P, C)


# speedup vs baseline: 3.1298x; 1.4066x over previous
import functools

import jax
import jax.numpy as jnp
from jax import lax
from jax.experimental import pallas as pl
from jax.experimental.pallas import tpu as pltpu
from jax.experimental.pallas import tpu_sc as plsc

_P = 7
_RPB = 4
_NW = 32


def _index_kernel(roi_ref, idx_ref, w_ref, *, H, W):
    roi = roi_ref[...]
    y1 = roi[:, 0][None, :]
    x1 = roi[:, 1][None, :]
    y2 = roi[:, 2][None, :]
    x2 = roi[:, 3][None, :]

    h = jnp.maximum(x2 - x1, 1)
    w = jnp.maximum(y2 - y1, 1)
    iv = lax.broadcasted_iota(jnp.int32, (_P, 1), 0).astype(jnp.float32)

    rpos = iv * (h.astype(jnp.float32) / _P)
    r0 = jnp.floor(rpos).astype(jnp.int32)
    r1 = jnp.minimum(r0 + 1, h - 1)
    rf = rpos - r0.astype(jnp.float32)
    R0 = jnp.clip(x1 + r0, 0, H - 1)
    R1 = jnp.clip(x1 + r1, 0, H - 1)

    cpos = iv * (w.astype(jnp.float32) / _P)
    c0 = jnp.floor(cpos).astype(jnp.int32)
    c1 = jnp.minimum(c0 + 1, w - 1)
    cf = cpos - c0.astype(jnp.float32)
    C0 = jnp.clip(y1 + c0, 0, W - 1)
    C1 = jnp.clip(y1 + c1, 0, W - 1)

    R0b = R0[:, None, :] * W
    R1b = R1[:, None, :] * W
    C0b = C0[None, :, :]
    C1b = C1[None, :, :]
    idx_ref[0] = R0b + C0b
    idx_ref[1] = R0b + C1b
    idx_ref[2] = R1b + C0b
    idx_ref[3] = R1b + C1b

    rfb = rf[:, None, :]
    cfb = cf[None, :, :]
    one = jnp.float32(1.0)
    w_ref[0] = (one - rfb) * (one - cfb)
    w_ref[1] = (one - rfb) * cfb
    w_ref[2] = rfb * (one - cfb)
    w_ref[3] = rfb * cfb


def _make_sc_kernel(C, NB):
    npos = _P * _P
    rows_per_block = _RPB * npos
    mesh = plsc.VectorSubcoreMesh(core_axis_name="c", subcore_axis_name="s")
    info = plsc.get_sparse_core_info()
    nc = info.num_cores

    @functools.partial(
        pl.kernel,
        mesh=mesh,
        out_type=jax.ShapeDtypeStruct((NB * rows_per_block, C), jnp.float32),
        scratch_types=[
            pltpu.VMEM((npos, 16), jnp.int32),
            pltpu.VMEM((npos, 16), jnp.float32),
            pltpu.VMEM((2, 4 * _RPB, C), jnp.float32),
            pltpu.VMEM((rows_per_block, C), jnp.float32),
            pltpu.SemaphoreType.DMA((2,)),
        ],
        compiler_params=pltpu.CompilerParams(use_tc_tiling_on_sc=False),
    )
    def sc_kernel(feat_hbm, idx_hbm, w_hbm, out_hbm, idx_v, w_v, gbuf, obuf, sem):
        wid = lax.axis_index("s") * nc + lax.axis_index("c")
        nblk = (NB - wid + _NW - 1) // _NW

        def outer(i, carry):
            b = wid + i * _NW
            pltpu.sync_copy(idx_hbm.at[b], idx_v)
            pltpu.sync_copy(w_hbm.at[b], w_v)

            pltpu.async_copy(feat_hbm.at[idx_v[0]], gbuf.at[0], sem.at[0])

            def inner(p, carry2):
                slot = lax.rem(p, 2)
                nslot = lax.rem(p + 1, 2)

                @pl.when(p + 1 < npos)
                def _prefetch():
                    pltpu.async_copy(feat_hbm.at[idx_v[p + 1]],
                                     gbuf.at[nslot], sem.at[nslot])

                pltpu.make_async_copy(feat_hbm.at[idx_v[p]],
                                      gbuf.at[slot], sem.at[slot]).wait()
                wrow = w_v[p]
                for r in range(_RPB):
                    w00 = wrow[r]
                    w01 = wrow[4 + r]
                    w10 = wrow[8 + r]
                    w11 = wrow[12 + r]
                    orow = r * npos + p

                    def cbody(c, carry3):
                        for u in range(4):
                            cs = pl.ds((c * 4 + u) * 16, 16)
                            obuf[orow, cs] = (w00 * gbuf[slot, r, cs]
                                              + w01 * gbuf[slot, 4 + r, cs]
                                              + w10 * gbuf[slot, 8 + r, cs]
                                              + w11 * gbuf[slot, 12 + r, cs])
                        return carry3

                    lax.fori_loop(0, C // 64, cbody, 0)
                return carry2

            lax.fori_loop(0, npos, inner, 0)
            pltpu.sync_copy(obuf, out_hbm.at[pl.ds(b * rows_per_block, rows_per_block)])
            return carry

        lax.fori_loop(0, nblk, outer, 0)

    return sc_kernel


def kernel(features, roi):
    B, H, W, C = features.shape
    N = roi.shape[0] * roi.shape[1]
    roi2 = roi.reshape(N, 4).astype(jnp.int32)
    feat2 = features.reshape(B * H * W, C)

    idx4, w4 = pl.pallas_call(
        functools.partial(_index_kernel, H=H, W=W),
        out_shape=[
            jax.ShapeDtypeStruct((4, _P, _P, N), jnp.int32),
            jax.ShapeDtypeStruct((4, _P, _P, N), jnp.float32),
        ],
    )(roi2)

    NB = N // _RPB
    npos = _P * _P
    idx_b = (idx4.reshape(4, npos, NB, _RPB)
             .transpose(2, 1, 0, 3)
             .reshape(NB, npos, 16))
    w_b = (w4.reshape(4, npos, NB, _RPB)
           .transpose(2, 1, 0, 3)
           .reshape(NB, npos, 16))

    out = _make_sc_kernel(C, NB)(feat2, idx_b, w_b)
    return out.reshape(N, _P, _P, C)


# speedup vs baseline: 3.1385x; 1.0028x over previous
import functools

import jax
import jax.numpy as jnp
from jax import lax
from jax.experimental import pallas as pl
from jax.experimental.pallas import tpu as pltpu
from jax.experimental.pallas import tpu_sc as plsc

_P = 7
_RPB = 16
_NW = 32


def _index_kernel(roi_ref, idx_ref, w_ref, *, H, W):
    roi = roi_ref[...]
    y1 = roi[:, 0][None, :]
    x1 = roi[:, 1][None, :]
    y2 = roi[:, 2][None, :]
    x2 = roi[:, 3][None, :]

    h = jnp.maximum(x2 - x1, 1)
    w = jnp.maximum(y2 - y1, 1)
    iv = lax.broadcasted_iota(jnp.int32, (_P, 1), 0).astype(jnp.float32)

    rpos = iv * (h.astype(jnp.float32) / _P)
    r0 = jnp.floor(rpos).astype(jnp.int32)
    r1 = jnp.minimum(r0 + 1, h - 1)
    rf = rpos - r0.astype(jnp.float32)
    R0 = jnp.clip(x1 + r0, 0, H - 1)
    R1 = jnp.clip(x1 + r1, 0, H - 1)

    cpos = iv * (w.astype(jnp.float32) / _P)
    c0 = jnp.floor(cpos).astype(jnp.int32)
    c1 = jnp.minimum(c0 + 1, w - 1)
    cf = cpos - c0.astype(jnp.float32)
    C0 = jnp.clip(y1 + c0, 0, W - 1)
    C1 = jnp.clip(y1 + c1, 0, W - 1)

    N = roi_ref.shape[0]
    npos = _P * _P

    def flat(x):
        return x.reshape(npos, N)

    R0b = R0[:, None, :] * W
    R1b = R1[:, None, :] * W
    C0b = C0[None, :, :]
    C1b = C1[None, :, :]
    idx_ref[0] = flat(R0b + C0b)
    idx_ref[1] = flat(R0b + C1b)
    idx_ref[2] = flat(R1b + C0b)
    idx_ref[3] = flat(R1b + C1b)

    rfb = jnp.broadcast_to(rf[:, None, :], (_P, _P, N))
    cfb = jnp.broadcast_to(cf[None, :, :], (_P, _P, N))
    one = jnp.float32(1.0)
    w_ref[0] = flat((one - rfb) * (one - cfb))
    w_ref[1] = flat((one - rfb) * cfb)
    w_ref[2] = flat(rfb * (one - cfb))
    w_ref[3] = flat(rfb * cfb)


def _make_sc_kernel(C, NB):
    npos = _P * _P
    mesh = plsc.VectorSubcoreMesh(core_axis_name="c", subcore_axis_name="s")
    info = plsc.get_sparse_core_info()
    nc = info.num_cores

    @functools.partial(
        pl.kernel,
        mesh=mesh,
        out_type=jax.ShapeDtypeStruct((NB * _RPB, npos, C), jnp.float32),
        scratch_types=[
            pltpu.VMEM((4, npos, 16), jnp.int32),
            pltpu.VMEM((4, npos, 16), jnp.float32),
            pltpu.VMEM((2, 4 * _RPB, C), jnp.float32),
            pltpu.VMEM((2, _RPB, C), jnp.float32),
            pltpu.SemaphoreType.DMA((2,)),
            pltpu.SemaphoreType.DMA((2,)),
        ],
        compiler_params=pltpu.CompilerParams(use_tc_tiling_on_sc=False),
    )
    def sc_kernel(feat_hbm, idx_hbm, w_hbm, out_hbm, idx_v, w_v, gbuf, obuf,
                  gsem, osem):
        wid = lax.axis_index("s") * nc + lax.axis_index("c")
        nblk = (NB - wid + _NW - 1) // _NW

        def fire(p, slot):
            for k in range(4):
                pltpu.async_copy(feat_hbm.at[idx_v[k, p]],
                                 gbuf.at[slot, pl.ds(k * _RPB, _RPB)],
                                 gsem.at[slot])

        def drain(p, slot):
            for k in range(4):
                pltpu.make_async_copy(feat_hbm.at[idx_v[k, p]],
                                      gbuf.at[slot, pl.ds(k * _RPB, _RPB)],
                                      gsem.at[slot]).wait()

        def outer(i, carry):
            b = wid + i * _NW
            base = b * _RPB
            pltpu.sync_copy(idx_hbm.at[:, :, pl.ds(base, _RPB)], idx_v)
            pltpu.sync_copy(w_hbm.at[:, :, pl.ds(base, _RPB)], w_v)
            fire(0, 0)

            def inner(p, carry2):
                slot = lax.rem(p, 2)
                nslot = lax.rem(p + 1, 2)

                @pl.when(p + 1 < npos)
                def _prefetch():
                    fire(p + 1, nslot)

                drain(p, slot)

                @pl.when(p >= 2)
                def _owait():
                    pltpu.make_async_copy(
                        obuf.at[slot],
                        out_hbm.at[pl.ds(base, _RPB), p - 2],
                        osem.at[slot]).wait()

                for r in range(_RPB):
                    w00 = w_v[0, p][r]
                    w01 = w_v[1, p][r]
                    w10 = w_v[2, p][r]
                    w11 = w_v[3, p][r]
                    for c in range(C // 16):
                        cs = pl.ds(c * 16, 16)
                        obuf[slot, r, cs] = (
                            w00 * gbuf[slot, r, cs]
                            + w01 * gbuf[slot, _RPB + r, cs]
                            + w10 * gbuf[slot, 2 * _RPB + r, cs]
                            + w11 * gbuf[slot, 3 * _RPB + r, cs])

                pltpu.async_copy(obuf.at[slot],
                                 out_hbm.at[pl.ds(base, _RPB), p],
                                 osem.at[slot])
                return carry2

            lax.fori_loop(0, npos, inner, 0)
            for p in (npos - 2, npos - 1):
                pltpu.make_async_copy(
                    obuf.at[p % 2],
                    out_hbm.at[pl.ds(base, _RPB), p],
                    osem.at[p % 2]).wait()
            return carry

        lax.fori_loop(0, nblk, outer, 0)

    return sc_kernel


def kernel(features, roi):
    B, H, W, C = features.shape
    N = roi.shape[0] * roi.shape[1]
    roi2 = roi.reshape(N, 4).astype(jnp.int32)
    feat2 = features.reshape(B * H * W, C)

    npos = _P * _P
    idx4, w4 = pl.pallas_call(
        functools.partial(_index_kernel, H=H, W=W),
        out_shape=[
            jax.ShapeDtypeStruct((4, npos, N), jnp.int32),
            jax.ShapeDtypeStruct((4, npos, N), jnp.float32),
        ],
    )(roi2)

    NB = N // _RPB
    out = _make_sc_kernel(C, NB)(feat2, idx4, w4)
    return out.reshape(N, _P, _P, C)


# speedup vs baseline: 3.2852x; 1.0467x over previous
import functools

import jax
import jax.numpy as jnp
from jax import lax
from jax.experimental import pallas as pl
from jax.experimental.pallas import tpu as pltpu
from jax.experimental.pallas import tpu_sc as plsc

_P = 7
_RPB = 16
_NW = 32


def _roi_t_kernel(roi_ref, out_ref):
    out_ref[...] = roi_ref[...].T


def _make_sc_kernel(H, W, C, NB):
    npos = _P * _P
    mesh = plsc.VectorSubcoreMesh(core_axis_name="c", subcore_axis_name="s")
    info = plsc.get_sparse_core_info()
    nc = info.num_cores
    fP = jnp.float32(_P)

    @functools.partial(
        pl.kernel,
        mesh=mesh,
        out_type=jax.ShapeDtypeStruct((NB * _RPB, npos, C), jnp.float32),
        scratch_types=[
            pltpu.VMEM((4, 16), jnp.int32),
            pltpu.VMEM((4, npos, 16), jnp.int32),
            pltpu.VMEM((4, npos, 16), jnp.float32),
            pltpu.VMEM((2, 4 * _RPB, C), jnp.float32),
            pltpu.VMEM((2, _RPB, C), jnp.float32),
            pltpu.SemaphoreType.DMA((2,)),
            pltpu.SemaphoreType.DMA((2,)),
        ],
        compiler_params=pltpu.CompilerParams(use_tc_tiling_on_sc=False),
    )
    def sc_kernel(feat_hbm, roit_hbm, out_hbm, roi_v, idx_v, w_v, gbuf, obuf,
                  gsem, osem):
        wid = lax.axis_index("s") * nc + lax.axis_index("c")
        nblk = (NB - wid + _NW - 1) // _NW

        def fire(p, slot):
            for k in range(4):
                pltpu.async_copy(feat_hbm.at[idx_v[k, p]],
                                 gbuf.at[slot, pl.ds(k * _RPB, _RPB)],
                                 gsem.at[slot])

        def drain(p, slot):
            for k in range(4):
                pltpu.make_async_copy(feat_hbm.at[idx_v[k, p]],
                                      gbuf.at[slot, pl.ds(k * _RPB, _RPB)],
                                      gsem.at[slot]).wait()

        def outer(i, carry):
            b = wid + i * _NW
            base = b * _RPB
            pltpu.sync_copy(roit_hbm.at[:, pl.ds(base, _RPB)], roi_v)
            y1 = roi_v[0]
            x1 = roi_v[1]
            y2 = roi_v[2]
            x2 = roi_v[3]
            one = jnp.float32(1.0)

            wd = jnp.maximum(y2 - y1, 1)
            wf7 = wd.astype(jnp.float32) / fP
            c0l, c1l, cfl, cf1l = [], [], [], []
            for j in range(_P):
                cpos = jnp.float32(j) * wf7
                c0 = cpos.astype(jnp.int32)
                c1 = jnp.minimum(c0 + 1, wd - 1)
                cf = cpos - c0.astype(jnp.float32)
                c0l.append(jnp.clip(y1 + c0, 0, W - 1))
                c1l.append(jnp.clip(y1 + c1, 0, W - 1))
                cfl.append(cf)
                cf1l.append(one - cf)

            h = jnp.maximum(x2 - x1, 1)
            hf7 = h.astype(jnp.float32) / fP
            for i in range(_P):
                rpos = jnp.float32(i) * hf7
                r0 = rpos.astype(jnp.int32)
                r1 = jnp.minimum(r0 + 1, h - 1)
                rf = rpos - r0.astype(jnp.float32)
                rf1 = one - rf
                R0W = jnp.clip(x1 + r0, 0, H - 1) * W
                R1W = jnp.clip(x1 + r1, 0, H - 1) * W
                for j in range(_P):
                    p = i * _P + j
                    idx_v[0, p] = R0W + c0l[j]
                    idx_v[1, p] = R0W + c1l[j]
                    idx_v[2, p] = R1W + c0l[j]
                    idx_v[3, p] = R1W + c1l[j]
                    w_v[0, p] = rf1 * cf1l[j]
                    w_v[1, p] = rf1 * cfl[j]
                    w_v[2, p] = rf * cf1l[j]
                    w_v[3, p] = rf * cfl[j]

            fire(0, 0)

            def inner(p, carry2):
                slot = lax.rem(p, 2)
                nslot = lax.rem(p + 1, 2)

                @pl.when(p + 1 < npos)
                def _prefetch():
                    fire(p + 1, nslot)

                drain(p, slot)

                @pl.when(p >= 2)
                def _owait():
                    pltpu.make_async_copy(
                        obuf.at[slot],
                        out_hbm.at[pl.ds(base, _RPB), p - 2],
                        osem.at[slot]).wait()

                for r in range(_RPB):
                    w00 = w_v[0, p][r]
                    w01 = w_v[1, p][r]
                    w10 = w_v[2, p][r]
                    w11 = w_v[3, p][r]
                    for c in range(C // 16):
                        cs = pl.ds(c * 16, 16)
                        obuf[slot, r, cs] = (
                            w00 * gbuf[slot, r, cs]
                            + w01 * gbuf[slot, _RPB + r, cs]
                            + w10 * gbuf[slot, 2 * _RPB + r, cs]
                            + w11 * gbuf[slot, 3 * _RPB + r, cs])

                pltpu.async_copy(obuf.at[slot],
                                 out_hbm.at[pl.ds(base, _RPB), p],
                                 osem.at[slot])
                return carry2

            lax.fori_loop(0, npos, inner, 0)
            for p in (npos - 2, npos - 1):
                pltpu.make_async_copy(
                    obuf.at[p % 2],
                    out_hbm.at[pl.ds(base, _RPB), p],
                    osem.at[p % 2]).wait()
            return carry

        lax.fori_loop(0, nblk, outer, 0)

    return sc_kernel


def kernel(features, roi):
    B, H, W, C = features.shape
    N = roi.shape[0] * roi.shape[1]
    roi2 = roi.reshape(N, 4).astype(jnp.int32)
    feat2 = features.reshape(B * H * W, C)

    roit = pl.pallas_call(
        _roi_t_kernel,
        out_shape=jax.ShapeDtypeStruct((4, N), jnp.int32),
    )(roi2)

    NB = N // _RPB
    out = _make_sc_kernel(H, W, C, NB)(feat2, roit)
    return out.reshape(N, _P, _P, C)
